# baseline (speedup 1.0000x reference)
import functools

import jax
import jax.numpy as jnp
from jax import lax
from jax.experimental import pallas as pl
from jax.experimental.pallas import tpu as pltpu

N_TASKS = 3
NUM_EXPERTS = 8
TOP_K = 2
BZS = 256
N_CAND = 4
B = N_CAND * BZS
KSPLIT = 8


def _bottom_gate_kernel(x_ref, fc1_ref, b1_ref, fc2_ref, b2_ref, wg_ref,
                        h_ref, gates_ref, aux_ref):
    r = pl.program_id(0)
    x = x_ref[...].astype(jnp.bfloat16)
    a0 = jnp.dot(x, fc1_ref[...], preferred_element_type=jnp.float32)
    a0 = jnp.maximum(a0 + b1_ref[...], 0.0).astype(jnp.bfloat16)
    h = jnp.dot(a0, fc2_ref[...], preferred_element_type=jnp.float32)
    h = h + b2_ref[...]
    hb = h.astype(jnp.bfloat16)
    h_ref[...] = hb
    gl_all = jnp.dot(hb, wg_ref[...], preferred_element_type=jnp.float32)
    iota = lax.broadcasted_iota(jnp.int32, (BZS, NUM_EXPERTS), 1)
    aux = jnp.float32(0.0)
    for j in range(N_TASKS):
        gl = gl_all[:, j * NUM_EXPERTS:(j + 1) * NUM_EXPERTS]
        m1 = jnp.max(gl, axis=1, keepdims=True)
        idx1 = jnp.min(jnp.where(gl == m1, iota, NUM_EXPERTS), axis=1,
                       keepdims=True)
        masked = jnp.where(iota == idx1, -jnp.inf, gl)
        m2 = jnp.max(masked, axis=1, keepdims=True)
        idx2 = jnp.min(jnp.where(masked == m2, iota, NUM_EXPERTS), axis=1,
                       keepdims=True)
        g1 = 1.0 / (1.0 + jnp.exp(m2 - m1))
        g2 = 1.0 - g1
        gates_j = (jnp.where(iota == idx1, g1, 0.0)
                   + jnp.where(iota == idx2, g2, 0.0))
        gates_ref[j] = gates_j
        imp = jnp.sum(gates_j, axis=0)
        mean = jnp.mean(imp)
        var = jnp.mean((imp - mean) ** 2)
        aux = aux + 0.01 * var / (mean * mean + 1e-10)

    @pl.when(r == 0)
    def _():
        aux_ref[...] = jnp.reshape(aux, (1, 1))

    @pl.when(r != 0)
    def _():
        aux_ref[...] += jnp.reshape(aux, (1, 1))


def _experts_kernel(h_ref, w1_ref, b1_ref, w2_ref, b2_ref, g_ref,
                    out_ref, acc_ref):
    e = pl.program_id(0)
    k = pl.program_id(1)
    w1b = w1_ref[...].astype(jnp.bfloat16)
    a1 = jnp.dot(h_ref[...], w1b, preferred_element_type=jnp.float32)
    a1 = jnp.maximum(a1 + b1_ref[...], 0.0).astype(jnp.bfloat16)
    w2b = w2_ref[...].astype(jnp.bfloat16)
    part = jnp.dot(a1, w2b, preferred_element_type=jnp.float32)

    @pl.when(k == 0)
    def _():
        acc_ref[...] = part

    @pl.when(k != 0)
    def _():
        acc_ref[...] += part

    last = k == KSPLIT - 1

    @pl.when(last & (e == 0))
    def _():
        acc = acc_ref[...] + b2_ref[...]
        for j in range(N_TASKS):
            out_ref[j] = g_ref[:, j:j + 1] * acc

    @pl.when(last & (e != 0))
    def _():
        acc = acc_ref[...] + b2_ref[...]
        for j in range(N_TASKS):
            out_ref[j] += g_ref[:, j:j + 1] * acc


def _tower_loss_kernel(moe_ref, tw1_ref, tb1_ref, tw2_ref, tb2_ref, s_ref,
                       aux_ref, preds_ref, loss_ref):
    j = pl.program_id(0)
    m = moe_ref[...].astype(jnp.bfloat16)
    t1 = jnp.dot(m, tw1_ref[...].astype(jnp.bfloat16),
                 preferred_element_type=jnp.float32)
    t1 = jnp.maximum(t1 + tb1_ref[...], 0.0)
    logits = jnp.sum(t1 * tw2_ref[...], axis=1, keepdims=True)
    logits = logits + tb2_ref[...]
    preds_ref[...] = 1.0 / (1.0 + jnp.exp(-logits))
    tot = jnp.float32(0.0)
    for i in range(N_CAND):
        s = s_ref[:, i:i + 1]
        labels = (s == jnp.max(s)).astype(jnp.float32)
        lg = logits[i * BZS:(i + 1) * BZS]
        bce = jnp.mean(jnp.maximum(lg, 0.0) - lg * labels
                       + jnp.log1p(jnp.exp(-jnp.abs(lg))))
        tot = tot + bce

    @pl.when(j == 0)
    def _():
        loss_ref[...] = aux_ref[...] + tot

    @pl.when(j != 0)
    def _():
        loss_ref[...] += tot

    @pl.when(j == N_TASKS - 1)
    def _():
        loss_ref[...] = loss_ref[...] / (N_CAND * N_TASKS)


@functools.partial(jax.jit, static_argnums=())
def kernel(cls_embed, scores, fc1_w, fc1_b, fc2_w, fc2_b, w_gate,
           exp_w1, exp_b1, exp_w2, exp_b2, tower_w1, tower_b1, tower_w2,
           tower_b2):
    f32 = jnp.float32
    bf16 = jnp.bfloat16
    isize = fc1_w.shape[0]
    hidden = fc1_w.shape[1]
    ehidden = exp_w1.shape[2]
    thidden = tower_w1.shape[2]
    kblk = ehidden // KSPLIT

    wg2 = jnp.transpose(w_gate, (1, 0, 2)).reshape(hidden,
                                                   N_TASKS * NUM_EXPERTS)
    x_all = jnp.transpose(cls_embed, (1, 0, 2)).reshape(B, isize)

    h, gates, aux = pl.pallas_call(
        _bottom_gate_kernel,
        grid=(N_CAND,),
        in_specs=[
            pl.BlockSpec((BZS, isize), lambda r: (r, 0)),
            pl.BlockSpec((isize, hidden), lambda r: (0, 0)),
            pl.BlockSpec((1, hidden), lambda r: (0, 0)),
            pl.BlockSpec((hidden, hidden), lambda r: (0, 0)),
            pl.BlockSpec((1, hidden), lambda r: (0, 0)),
            pl.BlockSpec((hidden, N_TASKS * NUM_EXPERTS), lambda r: (0, 0)),
        ],
        out_specs=(
            pl.BlockSpec((BZS, hidden), lambda r: (r, 0)),
            pl.BlockSpec((N_TASKS, BZS, NUM_EXPERTS), lambda r: (0, r, 0)),
            pl.BlockSpec((1, 1), lambda r: (0, 0)),
        ),
        out_shape=(
            jax.ShapeDtypeStruct((B, hidden), bf16),
            jax.ShapeDtypeStruct((N_TASKS, B, NUM_EXPERTS), f32),
            jax.ShapeDtypeStruct((1, 1), f32),
        ),
    )(x_all, fc1_w.astype(bf16), fc1_b.reshape(1, -1),
      fc2_w.astype(bf16), fc2_b.reshape(1, -1), wg2.astype(bf16))

    g_t = jnp.transpose(gates, (2, 1, 0))

    moe = pl.pallas_call(
        _experts_kernel,
        grid=(NUM_EXPERTS, KSPLIT),
        in_specs=[
            pl.BlockSpec((B, hidden), lambda e, k: (0, 0)),
            pl.BlockSpec((None, hidden, kblk), lambda e, k: (e, 0, k)),
            pl.BlockSpec((None, 1, kblk), lambda e, k: (e, 0, k)),
            pl.BlockSpec((None, kblk, hidden), lambda e, k: (e, k, 0)),
            pl.BlockSpec((None, 1, hidden), lambda e, k: (e, 0, 0)),
            pl.BlockSpec((None, B, N_TASKS), lambda e, k: (e, 0, 0)),
        ],
        out_specs=pl.BlockSpec((N_TASKS, B, hidden), lambda e, k: (0, 0, 0)),
        out_shape=jax.ShapeDtypeStruct((N_TASKS, B, hidden), f32),
        scratch_shapes=[pltpu.VMEM((B, hidden), f32)],
    )(h, exp_w1, exp_b1.reshape(NUM_EXPERTS, 1, ehidden), exp_w2,
      exp_b2.reshape(NUM_EXPERTS, 1, hidden), g_t)

    scores_t = jnp.transpose(scores, (1, 2, 0))

    preds, loss = pl.pallas_call(
        _tower_loss_kernel,
        grid=(N_TASKS,),
        in_specs=[
            pl.BlockSpec((None, B, hidden), lambda j: (j, 0, 0)),
            pl.BlockSpec((None, hidden, thidden), lambda j: (j, 0, 0)),
            pl.BlockSpec((None, 1, thidden), lambda j: (j, 0, 0)),
            pl.BlockSpec((None, 1, thidden), lambda j: (j, 0, 0)),
            pl.BlockSpec((None, 1, 1), lambda j: (j, 0, 0)),
            pl.BlockSpec((None, BZS, N_CAND), lambda j: (j, 0, 0)),
            pl.BlockSpec((1, 1), lambda j: (0, 0)),
        ],
        out_specs=(
            pl.BlockSpec((None, B, 1), lambda j: (j, 0, 0)),
            pl.BlockSpec((1, 1), lambda j: (0, 0)),
        ),
        out_shape=(
            jax.ShapeDtypeStruct((N_TASKS, B, 1), f32),
            jax.ShapeDtypeStruct((1, 1), f32),
        ),
    )(moe, tower_w1, tower_b1.reshape(N_TASKS, 1, thidden),
      tower_w2.reshape(N_TASKS, 1, thidden),
      tower_b2.reshape(N_TASKS, 1, 1), scores_t, aux)

    preds_out = preds.reshape(N_TASKS, N_CAND, BZS).transpose(1, 0, 2)
    return loss.reshape(()), preds_out


# speedup vs baseline: 1.0515x; 1.0515x over previous
import functools

import jax
import jax.numpy as jnp
from jax import lax
from jax.experimental import pallas as pl
from jax.experimental.pallas import tpu as pltpu

N_TASKS = 3
NUM_EXPERTS = 8
TOP_K = 2
BZS = 256
N_CAND = 4
B = N_CAND * BZS
CSPLIT = 8
CBLK = 2048 // CSPLIT


def _bottom_gate_kernel(x_ref, fc1_ref, b1_ref, fc2_ref, b2_ref, wg_ref,
                        h_ref, gates_ref, aux_ref):
    r = pl.program_id(0)
    x = x_ref[...].astype(jnp.bfloat16)
    a0 = jnp.dot(x, fc1_ref[...], preferred_element_type=jnp.float32)
    a0 = jnp.maximum(a0 + b1_ref[...], 0.0).astype(jnp.bfloat16)
    h = jnp.dot(a0, fc2_ref[...], preferred_element_type=jnp.float32)
    h = h + b2_ref[...]
    hb = h.astype(jnp.bfloat16)
    h_ref[...] = hb
    gl_all = jnp.dot(hb, wg_ref[...], preferred_element_type=jnp.float32)
    iota = lax.broadcasted_iota(jnp.int32, (BZS, NUM_EXPERTS), 1)
    aux = jnp.float32(0.0)
    for j in range(N_TASKS):
        gl = gl_all[:, j * NUM_EXPERTS:(j + 1) * NUM_EXPERTS]
        m1 = jnp.max(gl, axis=1, keepdims=True)
        idx1 = jnp.min(jnp.where(gl == m1, iota, NUM_EXPERTS), axis=1,
                       keepdims=True)
        masked = jnp.where(iota == idx1, -jnp.inf, gl)
        m2 = jnp.max(masked, axis=1, keepdims=True)
        idx2 = jnp.min(jnp.where(masked == m2, iota, NUM_EXPERTS), axis=1,
                       keepdims=True)
        g1 = 1.0 / (1.0 + jnp.exp(m2 - m1))
        g2 = 1.0 - g1
        gates_j = (jnp.where(iota == idx1, g1, 0.0)
                   + jnp.where(iota == idx2, g2, 0.0))
        gates_ref[j] = gates_j
        imp = jnp.sum(gates_j, axis=0)
        mean = jnp.mean(imp)
        var = jnp.mean((imp - mean) ** 2)
        aux = aux + 0.01 * var / (mean * mean + 1e-10)

    @pl.when(r == 0)
    def _():
        aux_ref[...] = jnp.reshape(aux, (1, 1))

    @pl.when(r != 0)
    def _():
        aux_ref[...] += jnp.reshape(aux, (1, 1))


def _experts_kernel(h_ref, w1_ref, b1_ref, w2_ref, b2_ref, g_ref,
                    out_ref, a1_ref):
    e = pl.program_id(0)
    c = pl.program_id(1)

    @pl.when(c == 0)
    def _():
        a1 = jnp.dot(h_ref[...], w1_ref[...],
                     preferred_element_type=jnp.float32)
        a1_ref[...] = jnp.maximum(a1 + b1_ref[...], 0.0).astype(jnp.bfloat16)

    part = jnp.dot(a1_ref[...], w2_ref[...],
                   preferred_element_type=jnp.float32)
    part = part + b2_ref[...]
    cols = pl.ds(c * CBLK, CBLK)

    @pl.when(e == 0)
    def _():
        for j in range(N_TASKS):
            out_ref[j, :, cols] = g_ref[:, j:j + 1] * part

    @pl.when(e != 0)
    def _():
        for j in range(N_TASKS):
            out_ref[j, :, cols] += g_ref[:, j:j + 1] * part


def _tower_loss_kernel(moe_ref, tw1_ref, tb1_ref, tw2_ref, tb2_ref, s_ref,
                       aux_ref, preds_ref, loss_ref):
    j = pl.program_id(0)
    m = moe_ref[...].astype(jnp.bfloat16)
    t1 = jnp.dot(m, tw1_ref[...].astype(jnp.bfloat16),
                 preferred_element_type=jnp.float32)
    t1 = jnp.maximum(t1 + tb1_ref[...], 0.0)
    logits = jnp.sum(t1 * tw2_ref[...], axis=1, keepdims=True)
    logits = logits + tb2_ref[...]
    preds_ref[...] = 1.0 / (1.0 + jnp.exp(-logits))
    tot = jnp.float32(0.0)
    for i in range(N_CAND):
        s = s_ref[:, i:i + 1]
        labels = (s == jnp.max(s)).astype(jnp.float32)
        lg = logits[i * BZS:(i + 1) * BZS]
        bce = jnp.mean(jnp.maximum(lg, 0.0) - lg * labels
                       + jnp.log1p(jnp.exp(-jnp.abs(lg))))
        tot = tot + bce

    @pl.when(j == 0)
    def _():
        loss_ref[...] = aux_ref[...] + tot

    @pl.when(j != 0)
    def _():
        loss_ref[...] += tot

    @pl.when(j == N_TASKS - 1)
    def _():
        loss_ref[...] = loss_ref[...] / (N_CAND * N_TASKS)


@functools.partial(jax.jit, static_argnums=())
def kernel(cls_embed, scores, fc1_w, fc1_b, fc2_w, fc2_b, w_gate,
           exp_w1, exp_b1, exp_w2, exp_b2, tower_w1, tower_b1, tower_w2,
           tower_b2):
    f32 = jnp.float32
    bf16 = jnp.bfloat16
    isize = fc1_w.shape[0]
    hidden = fc1_w.shape[1]
    ehidden = exp_w1.shape[2]
    thidden = tower_w1.shape[2]

    wg2 = jnp.transpose(w_gate, (1, 0, 2)).reshape(hidden,
                                                   N_TASKS * NUM_EXPERTS)
    x_all = jnp.transpose(cls_embed, (1, 0, 2)).reshape(B, isize)

    h, gates, aux = pl.pallas_call(
        _bottom_gate_kernel,
        grid=(N_CAND,),
        in_specs=[
            pl.BlockSpec((BZS, isize), lambda r: (r, 0)),
            pl.BlockSpec((isize, hidden), lambda r: (0, 0)),
            pl.BlockSpec((1, hidden), lambda r: (0, 0)),
            pl.BlockSpec((hidden, hidden), lambda r: (0, 0)),
            pl.BlockSpec((1, hidden), lambda r: (0, 0)),
            pl.BlockSpec((hidden, N_TASKS * NUM_EXPERTS), lambda r: (0, 0)),
        ],
        out_specs=(
            pl.BlockSpec((BZS, hidden), lambda r: (r, 0)),
            pl.BlockSpec((N_TASKS, BZS, NUM_EXPERTS), lambda r: (0, r, 0)),
            pl.BlockSpec((1, 1), lambda r: (0, 0)),
        ),
        out_shape=(
            jax.ShapeDtypeStruct((B, hidden), bf16),
            jax.ShapeDtypeStruct((N_TASKS, B, NUM_EXPERTS), f32),
            jax.ShapeDtypeStruct((1, 1), f32),
        ),
    )(x_all, fc1_w.astype(bf16), fc1_b.reshape(1, -1),
      fc2_w.astype(bf16), fc2_b.reshape(1, -1), wg2.astype(bf16))

    g_t = jnp.transpose(gates, (2, 1, 0))

    moe = pl.pallas_call(
        _experts_kernel,
        grid=(NUM_EXPERTS, CSPLIT),
        in_specs=[
            pl.BlockSpec((B, hidden), lambda e, c: (0, 0)),
            pl.BlockSpec((None, hidden, ehidden), lambda e, c: (e, 0, 0)),
            pl.BlockSpec((None, 1, ehidden), lambda e, c: (e, 0, 0)),
            pl.BlockSpec((None, ehidden, CBLK), lambda e, c: (e, 0, c)),
            pl.BlockSpec((None, 1, CBLK), lambda e, c: (e, 0, c)),
            pl.BlockSpec((None, B, N_TASKS), lambda e, c: (e, 0, 0)),
        ],
        out_specs=pl.BlockSpec((N_TASKS, B, hidden), lambda e, c: (0, 0, 0)),
        out_shape=jax.ShapeDtypeStruct((N_TASKS, B, hidden), f32),
        scratch_shapes=[pltpu.VMEM((B, ehidden), bf16)],
    )(h, exp_w1.astype(bf16), exp_b1.reshape(NUM_EXPERTS, 1, ehidden),
      exp_w2.astype(bf16), exp_b2.reshape(NUM_EXPERTS, 1, hidden), g_t)

    scores_t = jnp.transpose(scores, (1, 2, 0))

    preds, loss = pl.pallas_call(
        _tower_loss_kernel,
        grid=(N_TASKS,),
        in_specs=[
            pl.BlockSpec((None, B, hidden), lambda j: (j, 0, 0)),
            pl.BlockSpec((None, hidden, thidden), lambda j: (j, 0, 0)),
            pl.BlockSpec((None, 1, thidden), lambda j: (j, 0, 0)),
            pl.BlockSpec((None, 1, thidden), lambda j: (j, 0, 0)),
            pl.BlockSpec((None, 1, 1), lambda j: (j, 0, 0)),
            pl.BlockSpec((None, BZS, N_CAND), lambda j: (j, 0, 0)),
            pl.BlockSpec((1, 1), lambda j: (0, 0)),
        ],
        out_specs=(
            pl.BlockSpec((None, B, 1), lambda j: (j, 0, 0)),
            pl.BlockSpec((1, 1), lambda j: (0, 0)),
        ),
        out_shape=(
            jax.ShapeDtypeStruct((N_TASKS, B, 1), f32),
            jax.ShapeDtypeStruct((1, 1), f32),
        ),
    )(moe, tower_w1, tower_b1.reshape(N_TASKS, 1, thidden),
      tower_w2.reshape(N_TASKS, 1, thidden),
      tower_b2.reshape(N_TASKS, 1, 1), scores_t, aux)

    preds_out = preds.reshape(N_TASKS, N_CAND, BZS).transpose(1, 0, 2)
    return loss.reshape(()), preds_out


# speedup vs baseline: 1.3854x; 1.3176x over previous
import functools

import jax
import jax.numpy as jnp
from jax import lax
from jax.experimental import pallas as pl
from jax.experimental.pallas import tpu as pltpu

N_TASKS = 3
NUM_EXPERTS = 8
TOP_K = 2
BZS = 256
N_CAND = 4
B = N_CAND * BZS
CSPLIT = 8
CBLK = 2048 // CSPLIT
KSPLIT = 4
KBLK = 2048 // KSPLIT


def _bottom_gate_kernel(x_ref, fc1_ref, b1_ref, fc2_ref, b2_ref, wg_ref,
                        h_ref, gates_ref, aux_ref):
    r = pl.program_id(0)
    x = x_ref[...].astype(jnp.bfloat16)
    a0 = jnp.dot(x, fc1_ref[...], preferred_element_type=jnp.float32)
    a0 = jnp.maximum(a0 + b1_ref[...], 0.0).astype(jnp.bfloat16)
    h = jnp.dot(a0, fc2_ref[...], preferred_element_type=jnp.float32)
    h = h + b2_ref[...]
    hb = h.astype(jnp.bfloat16)
    h_ref[...] = hb
    gl_all = jnp.dot(hb, wg_ref[...], preferred_element_type=jnp.float32)
    iota = lax.broadcasted_iota(jnp.int32, (BZS, NUM_EXPERTS), 1)
    aux = jnp.float32(0.0)
    for j in range(N_TASKS):
        gl = gl_all[:, j * NUM_EXPERTS:(j + 1) * NUM_EXPERTS]
        m1 = jnp.max(gl, axis=1, keepdims=True)
        idx1 = jnp.min(jnp.where(gl == m1, iota, NUM_EXPERTS), axis=1,
                       keepdims=True)
        masked = jnp.where(iota == idx1, -jnp.inf, gl)
        m2 = jnp.max(masked, axis=1, keepdims=True)
        idx2 = jnp.min(jnp.where(masked == m2, iota, NUM_EXPERTS), axis=1,
                       keepdims=True)
        g1 = 1.0 / (1.0 + jnp.exp(m2 - m1))
        g2 = 1.0 - g1
        gates_j = (jnp.where(iota == idx1, g1, 0.0)
                   + jnp.where(iota == idx2, g2, 0.0))
        gates_ref[j] = gates_j
        imp = jnp.sum(gates_j, axis=0)
        mean = jnp.mean(imp)
        var = jnp.mean((imp - mean) ** 2)
        aux = aux + 0.01 * var / (mean * mean + 1e-10)

    @pl.when(r == 0)
    def _():
        aux_ref[...] = jnp.reshape(aux, (1, 1))

    @pl.when(r != 0)
    def _():
        aux_ref[...] += jnp.reshape(aux, (1, 1))


def _experts_kernel(h_ref, w1_ref, b1_ref, w2_ref, b2_ref, g_ref,
                    out_ref, a1f_ref, a1b_ref):
    e = pl.program_id(0)
    s = pl.program_id(1)

    @pl.when(s == 0)
    def _():
        a1f_ref[...] = jnp.dot(h_ref[...], w1_ref[...].astype(jnp.bfloat16),
                               preferred_element_type=jnp.float32)

    @pl.when((s > 0) & (s < KSPLIT))
    def _():
        a1f_ref[...] += jnp.dot(h_ref[...], w1_ref[...].astype(jnp.bfloat16),
                                preferred_element_type=jnp.float32)

    @pl.when(s == KSPLIT - 1)
    def _():
        a1b_ref[...] = jnp.maximum(a1f_ref[...] + b1_ref[...],
                                   0.0).astype(jnp.bfloat16)

    @pl.when(s >= KSPLIT)
    def _():
        part = jnp.dot(a1b_ref[...], w2_ref[...].astype(jnp.bfloat16),
                       preferred_element_type=jnp.float32)
        part = part + b2_ref[...]
        cols = pl.ds((s - KSPLIT) * CBLK, CBLK)

        @pl.when(e == 0)
        def _():
            for j in range(N_TASKS):
                out_ref[j, :, cols] = g_ref[:, j:j + 1] * part

        @pl.when(e != 0)
        def _():
            for j in range(N_TASKS):
                out_ref[j, :, cols] += g_ref[:, j:j + 1] * part


def _tower_loss_kernel(moe_ref, tw1_ref, tb1_ref, tw2_ref, tb2_ref, s_ref,
                       aux_ref, preds_ref, loss_ref):
    j = pl.program_id(0)
    m = moe_ref[...].astype(jnp.bfloat16)
    t1 = jnp.dot(m, tw1_ref[...].astype(jnp.bfloat16),
                 preferred_element_type=jnp.float32)
    t1 = jnp.maximum(t1 + tb1_ref[...], 0.0)
    logits = jnp.sum(t1 * tw2_ref[...], axis=1, keepdims=True)
    logits = logits + tb2_ref[...]
    preds_ref[...] = 1.0 / (1.0 + jnp.exp(-logits))
    tot = jnp.float32(0.0)
    for i in range(N_CAND):
        s = s_ref[:, i:i + 1]
        labels = (s == jnp.max(s)).astype(jnp.float32)
        lg = logits[i * BZS:(i + 1) * BZS]
        bce = jnp.mean(jnp.maximum(lg, 0.0) - lg * labels
                       + jnp.log1p(jnp.exp(-jnp.abs(lg))))
        tot = tot + bce

    @pl.when(j == 0)
    def _():
        loss_ref[...] = aux_ref[...] + tot

    @pl.when(j != 0)
    def _():
        loss_ref[...] += tot

    @pl.when(j == N_TASKS - 1)
    def _():
        loss_ref[...] = loss_ref[...] / (N_CAND * N_TASKS)


@functools.partial(jax.jit, static_argnums=())
def kernel(cls_embed, scores, fc1_w, fc1_b, fc2_w, fc2_b, w_gate,
           exp_w1, exp_b1, exp_w2, exp_b2, tower_w1, tower_b1, tower_w2,
           tower_b2):
    f32 = jnp.float32
    bf16 = jnp.bfloat16
    isize = fc1_w.shape[0]
    hidden = fc1_w.shape[1]
    ehidden = exp_w1.shape[2]
    thidden = tower_w1.shape[2]

    wg2 = jnp.transpose(w_gate, (1, 0, 2)).reshape(hidden,
                                                   N_TASKS * NUM_EXPERTS)
    x_all = jnp.transpose(cls_embed, (1, 0, 2)).reshape(B, isize)

    h, gates, aux = pl.pallas_call(
        _bottom_gate_kernel,
        grid=(N_CAND,),
        in_specs=[
            pl.BlockSpec((BZS, isize), lambda r: (r, 0)),
            pl.BlockSpec((isize, hidden), lambda r: (0, 0)),
            pl.BlockSpec((1, hidden), lambda r: (0, 0)),
            pl.BlockSpec((hidden, hidden), lambda r: (0, 0)),
            pl.BlockSpec((1, hidden), lambda r: (0, 0)),
            pl.BlockSpec((hidden, N_TASKS * NUM_EXPERTS), lambda r: (0, 0)),
        ],
        out_specs=(
            pl.BlockSpec((BZS, hidden), lambda r: (r, 0)),
            pl.BlockSpec((N_TASKS, BZS, NUM_EXPERTS), lambda r: (0, r, 0)),
            pl.BlockSpec((1, 1), lambda r: (0, 0)),
        ),
        out_shape=(
            jax.ShapeDtypeStruct((B, hidden), bf16),
            jax.ShapeDtypeStruct((N_TASKS, B, NUM_EXPERTS), f32),
            jax.ShapeDtypeStruct((1, 1), f32),
        ),
    )(x_all, fc1_w.astype(bf16), fc1_b.reshape(1, -1),
      fc2_w.astype(bf16), fc2_b.reshape(1, -1), wg2.astype(bf16))

    g_t = jnp.transpose(gates, (2, 1, 0))

    moe = pl.pallas_call(
        _experts_kernel,
        grid=(NUM_EXPERTS, KSPLIT + CSPLIT),
        in_specs=[
            pl.BlockSpec((B, KBLK),
                         lambda e, s: (0, jnp.minimum(s, KSPLIT - 1))),
            pl.BlockSpec((None, KBLK, ehidden),
                         lambda e, s: (e, jnp.minimum(s, KSPLIT - 1), 0)),
            pl.BlockSpec((None, 1, ehidden), lambda e, s: (e, 0, 0)),
            pl.BlockSpec((None, ehidden, CBLK),
                         lambda e, s: (e, 0, jnp.maximum(s - KSPLIT, 0))),
            pl.BlockSpec((None, 1, CBLK),
                         lambda e, s: (e, 0, jnp.maximum(s - KSPLIT, 0))),
            pl.BlockSpec((None, B, N_TASKS), lambda e, s: (e, 0, 0)),
        ],
        out_specs=pl.BlockSpec((N_TASKS, B, hidden), lambda e, s: (0, 0, 0)),
        out_shape=jax.ShapeDtypeStruct((N_TASKS, B, hidden), f32),
        scratch_shapes=[pltpu.VMEM((B, ehidden), f32),
                        pltpu.VMEM((B, ehidden), bf16)],
    )(h, exp_w1, exp_b1.reshape(NUM_EXPERTS, 1, ehidden),
      exp_w2, exp_b2.reshape(NUM_EXPERTS, 1, hidden), g_t)

    scores_t = jnp.transpose(scores, (1, 2, 0))

    preds, loss = pl.pallas_call(
        _tower_loss_kernel,
        grid=(N_TASKS,),
        in_specs=[
            pl.BlockSpec((None, B, hidden), lambda j: (j, 0, 0)),
            pl.BlockSpec((None, hidden, thidden), lambda j: (j, 0, 0)),
            pl.BlockSpec((None, 1, thidden), lambda j: (j, 0, 0)),
            pl.BlockSpec((None, 1, thidden), lambda j: (j, 0, 0)),
            pl.BlockSpec((None, 1, 1), lambda j: (j, 0, 0)),
            pl.BlockSpec((None, BZS, N_CAND), lambda j: (j, 0, 0)),
            pl.BlockSpec((1, 1), lambda j: (0, 0)),
        ],
        out_specs=(
            pl.BlockSpec((None, B, 1), lambda j: (j, 0, 0)),
            pl.BlockSpec((1, 1), lambda j: (0, 0)),
        ),
        out_shape=(
            jax.ShapeDtypeStruct((N_TASKS, B, 1), f32),
            jax.ShapeDtypeStruct((1, 1), f32),
        ),
    )(moe, tower_w1, tower_b1.reshape(N_TASKS, 1, thidden),
      tower_w2.reshape(N_TASKS, 1, thidden),
      tower_b2.reshape(N_TASKS, 1, 1), scores_t, aux)

    preds_out = preds.reshape(N_TASKS, N_CAND, BZS).transpose(1, 0, 2)
    return loss.reshape(()), preds_out


# speedup vs baseline: 1.4465x; 1.0441x over previous
import functools

import jax
import jax.numpy as jnp
from jax import lax
from jax.experimental import pallas as pl
from jax.experimental.pallas import tpu as pltpu

N_TASKS = 3
NUM_EXPERTS = 8
TOP_K = 2
BZS = 256
N_CAND = 4
B = N_CAND * BZS
CSPLIT = 8
CBLK = 2048 // CSPLIT
KSPLIT = 4
KBLK = 2048 // KSPLIT


def _bottom_gate_kernel(x_ref, fc1_ref, b1_ref, fc2_ref, b2_ref, wg_ref,
                        h_ref, gates_ref, aux_ref):
    r = pl.program_id(0)
    x = x_ref[...].astype(jnp.bfloat16)
    a0 = jnp.dot(x, fc1_ref[...].astype(jnp.bfloat16),
                 preferred_element_type=jnp.float32)
    a0 = jnp.maximum(a0 + b1_ref[...], 0.0).astype(jnp.bfloat16)
    h = jnp.dot(a0, fc2_ref[...].astype(jnp.bfloat16),
                preferred_element_type=jnp.float32)
    h = h + b2_ref[...]
    hb = h.astype(jnp.bfloat16)
    h_ref[...] = hb
    gl_all = jnp.dot(hb, wg_ref[...].astype(jnp.bfloat16),
                     preferred_element_type=jnp.float32)
    iota = lax.broadcasted_iota(jnp.int32, (BZS, NUM_EXPERTS), 1)
    aux = jnp.float32(0.0)
    for j in range(N_TASKS):
        gl = gl_all[:, j * NUM_EXPERTS:(j + 1) * NUM_EXPERTS]
        m1 = jnp.max(gl, axis=1, keepdims=True)
        idx1 = jnp.min(jnp.where(gl == m1, iota, NUM_EXPERTS), axis=1,
                       keepdims=True)
        masked = jnp.where(iota == idx1, -jnp.inf, gl)
        m2 = jnp.max(masked, axis=1, keepdims=True)
        idx2 = jnp.min(jnp.where(masked == m2, iota, NUM_EXPERTS), axis=1,
                       keepdims=True)
        g1 = 1.0 / (1.0 + jnp.exp(m2 - m1))
        g2 = 1.0 - g1
        gates_j = (jnp.where(iota == idx1, g1, 0.0)
                   + jnp.where(iota == idx2, g2, 0.0))
        gates_ref[j] = gates_j
        imp = jnp.sum(gates_j, axis=0)
        mean = jnp.mean(imp)
        var = jnp.mean((imp - mean) ** 2)
        aux = aux + 0.01 * var / (mean * mean + 1e-10)

    @pl.when(r == 0)
    def _():
        aux_ref[...] = jnp.reshape(aux, (1, 1))

    @pl.when(r != 0)
    def _():
        aux_ref[...] += jnp.reshape(aux, (1, 1))


def _experts_kernel(h_ref, w1_ref, b1_ref, w2_ref, b2_ref, g_ref,
                    out_ref, a1f_ref, a1b_ref):
    e = pl.program_id(0)
    s = pl.program_id(1)

    @pl.when(s == 0)
    def _():
        a1f_ref[...] = jnp.dot(h_ref[...], w1_ref[...].astype(jnp.bfloat16),
                               preferred_element_type=jnp.float32)

    @pl.when((s > 0) & (s < KSPLIT))
    def _():
        a1f_ref[...] += jnp.dot(h_ref[...], w1_ref[...].astype(jnp.bfloat16),
                                preferred_element_type=jnp.float32)

    @pl.when(s == KSPLIT - 1)
    def _():
        a1b_ref[...] = jnp.maximum(a1f_ref[...] + b1_ref[...],
                                   0.0).astype(jnp.bfloat16)

    @pl.when(s >= KSPLIT)
    def _():
        part = jnp.dot(a1b_ref[...], w2_ref[...].astype(jnp.bfloat16),
                       preferred_element_type=jnp.float32)
        part = part + b2_ref[...]
        cols = pl.ds((s - KSPLIT) * CBLK, CBLK)

        @pl.when(e == 0)
        def _():
            for j in range(N_TASKS):
                out_ref[j, :, cols] = g_ref[:, j:j + 1] * part

        @pl.when(e != 0)
        def _():
            for j in range(N_TASKS):
                out_ref[j, :, cols] += g_ref[:, j:j + 1] * part


def _tower_loss_kernel(moe_ref, tw1_ref, tb1_ref, tw2_ref, tb2_ref, s_ref,
                       aux_ref, preds_ref, loss_ref):
    j = pl.program_id(0)
    m = moe_ref[...].astype(jnp.bfloat16)
    t1 = jnp.dot(m, tw1_ref[...].astype(jnp.bfloat16),
                 preferred_element_type=jnp.float32)
    t1 = jnp.maximum(t1 + tb1_ref[...], 0.0)
    logits = jnp.sum(t1 * tw2_ref[...], axis=1, keepdims=True)
    logits = logits + tb2_ref[...]
    preds_ref[...] = 1.0 / (1.0 + jnp.exp(-logits))
    tot = jnp.float32(0.0)
    for i in range(N_CAND):
        s = s_ref[:, i:i + 1]
        labels = (s == jnp.max(s)).astype(jnp.float32)
        lg = logits[i * BZS:(i + 1) * BZS]
        bce = jnp.mean(jnp.maximum(lg, 0.0) - lg * labels
                       + jnp.log1p(jnp.exp(-jnp.abs(lg))))
        tot = tot + bce

    @pl.when(j == 0)
    def _():
        loss_ref[...] = aux_ref[...] + tot

    @pl.when(j != 0)
    def _():
        loss_ref[...] += tot

    @pl.when(j == N_TASKS - 1)
    def _():
        loss_ref[...] = loss_ref[...] / (N_CAND * N_TASKS)


@functools.partial(jax.jit, static_argnums=())
def kernel(cls_embed, scores, fc1_w, fc1_b, fc2_w, fc2_b, w_gate,
           exp_w1, exp_b1, exp_w2, exp_b2, tower_w1, tower_b1, tower_w2,
           tower_b2):
    f32 = jnp.float32
    bf16 = jnp.bfloat16
    isize = fc1_w.shape[0]
    hidden = fc1_w.shape[1]
    ehidden = exp_w1.shape[2]
    thidden = tower_w1.shape[2]

    wg2 = jnp.transpose(w_gate, (1, 0, 2)).reshape(hidden,
                                                   N_TASKS * NUM_EXPERTS)
    x_all = jnp.transpose(cls_embed, (1, 0, 2)).reshape(B, isize)

    h, gates, aux = pl.pallas_call(
        _bottom_gate_kernel,
        grid=(N_CAND,),
        in_specs=[
            pl.BlockSpec((BZS, isize), lambda r: (r, 0)),
            pl.BlockSpec((isize, hidden), lambda r: (0, 0)),
            pl.BlockSpec((1, hidden), lambda r: (0, 0)),
            pl.BlockSpec((hidden, hidden), lambda r: (0, 0)),
            pl.BlockSpec((1, hidden), lambda r: (0, 0)),
            pl.BlockSpec((hidden, N_TASKS * NUM_EXPERTS), lambda r: (0, 0)),
        ],
        out_specs=(
            pl.BlockSpec((BZS, hidden), lambda r: (r, 0)),
            pl.BlockSpec((N_TASKS, BZS, NUM_EXPERTS), lambda r: (0, r, 0)),
            pl.BlockSpec((1, 1), lambda r: (0, 0)),
        ),
        out_shape=(
            jax.ShapeDtypeStruct((B, hidden), bf16),
            jax.ShapeDtypeStruct((N_TASKS, B, NUM_EXPERTS), f32),
            jax.ShapeDtypeStruct((1, 1), f32),
        ),
    )(x_all, fc1_w, fc1_b.reshape(1, -1),
      fc2_w, fc2_b.reshape(1, -1), wg2)

    g_t = jnp.transpose(gates, (2, 1, 0))

    moe = pl.pallas_call(
        _experts_kernel,
        grid=(NUM_EXPERTS, KSPLIT + CSPLIT),
        in_specs=[
            pl.BlockSpec((B, KBLK),
                         lambda e, s: (0, jnp.minimum(s, KSPLIT - 1))),
            pl.BlockSpec((None, KBLK, ehidden),
                         lambda e, s: (e, jnp.minimum(s, KSPLIT - 1), 0)),
            pl.BlockSpec((None, 1, ehidden), lambda e, s: (e, 0, 0)),
            pl.BlockSpec((None, ehidden, CBLK),
                         lambda e, s: (e, 0, jnp.maximum(s - KSPLIT, 0))),
            pl.BlockSpec((None, 1, CBLK),
                         lambda e, s: (e, 0, jnp.maximum(s - KSPLIT, 0))),
            pl.BlockSpec((None, B, N_TASKS), lambda e, s: (e, 0, 0)),
        ],
        out_specs=pl.BlockSpec((N_TASKS, B, hidden), lambda e, s: (0, 0, 0)),
        out_shape=jax.ShapeDtypeStruct((N_TASKS, B, hidden), f32),
        scratch_shapes=[pltpu.VMEM((B, ehidden), f32),
                        pltpu.VMEM((B, ehidden), bf16)],
    )(h, exp_w1, exp_b1.reshape(NUM_EXPERTS, 1, ehidden),
      exp_w2, exp_b2.reshape(NUM_EXPERTS, 1, hidden), g_t)

    scores_t = jnp.transpose(scores, (1, 2, 0))

    preds, loss = pl.pallas_call(
        _tower_loss_kernel,
        grid=(N_TASKS,),
        in_specs=[
            pl.BlockSpec((None, B, hidden), lambda j: (j, 0, 0)),
            pl.BlockSpec((None, hidden, thidden), lambda j: (j, 0, 0)),
            pl.BlockSpec((None, 1, thidden), lambda j: (j, 0, 0)),
            pl.BlockSpec((None, 1, thidden), lambda j: (j, 0, 0)),
            pl.BlockSpec((None, 1, 1), lambda j: (j, 0, 0)),
            pl.BlockSpec((None, BZS, N_CAND), lambda j: (j, 0, 0)),
            pl.BlockSpec((1, 1), lambda j: (0, 0)),
        ],
        out_specs=(
            pl.BlockSpec((None, B, 1), lambda j: (j, 0, 0)),
            pl.BlockSpec((1, 1), lambda j: (0, 0)),
        ),
        out_shape=(
            jax.ShapeDtypeStruct((N_TASKS, B, 1), f32),
            jax.ShapeDtypeStruct((1, 1), f32),
        ),
    )(moe, tower_w1, tower_b1.reshape(N_TASKS, 1, thidden),
      tower_w2.reshape(N_TASKS, 1, thidden),
      tower_b2.reshape(N_TASKS, 1, 1), scores_t, aux)

    preds_out = preds.reshape(N_TASKS, N_CAND, BZS).transpose(1, 0, 2)
    return loss.reshape(()), preds_out
